# baseline (speedup 1.0000x reference)
import functools

import jax
import jax.numpy as jnp
from jax import lax
from jax.experimental import pallas as pl
from jax.experimental.pallas import tpu as pltpu
from jax.experimental.pallas import tpu_sc as plsc

SEQ = 8192
D = 768
BATCH = 4
TOTAL = BATCH * SEQ
NC, NS = 2, 16
NW = NC * NS
PER_W = TOTAL // NW
W_PER_B = SEQ // PER_W
CH = 32
NCHUNK = PER_W // CH
NBUF = 4


def _build():
    mesh = plsc.VectorSubcoreMesh(core_axis_name="c", subcore_axis_name="s")

    @functools.partial(
        pl.kernel,
        mesh=mesh,
        out_type=jax.ShapeDtypeStruct((BATCH, SEQ, D), jnp.float32),
        scratch_types=[
            pltpu.VMEM((PER_W,), jnp.int32),
        ] + [pltpu.VMEM((CH, D), jnp.float32)] * NBUF
          + [pltpu.SemaphoreType.DMA] * (2 * NBUF),
    )
    def gather_kernel(idx_hbm, table_hbm, out_hbm, idx_v, *rest):
        bufs = rest[:NBUF]
        gsems = rest[NBUF:2 * NBUF]
        wsems = rest[2 * NBUF:]
        wid = lax.axis_index("s") * NC + lax.axis_index("c")
        bb = wid // W_PER_B
        s0 = (wid % W_PER_B) * PER_W
        pltpu.sync_copy(idx_hbm.at[bb, pl.ds(s0, PER_W)], idx_v)

        gcp = [None] * NBUF
        wcp = [None] * NBUF
        for b in range(NBUF):
            gcp[b] = pltpu.async_copy(
                table_hbm.at[idx_v.at[pl.ds(b * CH, CH)]], bufs[b], gsems[b])
        for j in range(NCHUNK):
            b = j % NBUF
            gcp[b].wait()
            wcp[b] = pltpu.async_copy(
                bufs[b], out_hbm.at[bb, pl.ds(s0 + j * CH, CH)], wsems[b])
            if j + NBUF < NCHUNK:
                wcp[b].wait()
                gcp[b] = pltpu.async_copy(
                    table_hbm.at[idx_v.at[pl.ds((j + NBUF) * CH, CH)]],
                    bufs[b], gsems[b])
        for b in range(NBUF):
            wcp[b].wait()

    return gather_kernel


_gather = _build()


@jax.jit
def kernel(t, weight):
    return _gather(t.astype(jnp.int32), weight)
